# baseline (speedup 1.0000x reference)
import functools

import jax
import jax.numpy as jnp
from jax import lax
from jax.experimental import pallas as pl
from jax.experimental.pallas import tpu as pltpu
from jax.experimental.pallas import tpu_sc as plsc

_ROWS = 32768
_HID = 1024
_BLOCK = 2048

_NC = 2
_NS = 16
_NW = _NC * _NS
_WROWS = _ROWS // _NW
_BROWS = 32
_NDMA = _WROWS // _BROWS


def _logits_body(hs_ref, gw_ref, logits_ref):
    logits_ref[...] = jax.lax.dot_general(
        hs_ref[...], gw_ref[...],
        dimension_numbers=(((1,), (1,)), ((), ())),
        preferred_element_type=jnp.float32,
    )


_SROWS = 512


def _zero_fill_body(out_hbm, buf, shared, sem):
    sid = lax.axis_index("s")
    wid = sid * _NC + lax.axis_index("c")
    b = wid // 8
    row0 = (wid % 8) * _WROWS

    def zero_row(r, carry):
        def zero_chunk(c, inner):
            buf[r, pl.ds(c * 16, 16)] = jnp.zeros((16,), jnp.float32)
            return inner

        return lax.fori_loop(0, _HID // 16, zero_chunk, carry)

    lax.fori_loop(0, _BROWS, zero_row, 0)

    pltpu.sync_copy(buf, shared.at[pl.ds(sid * _BROWS, _BROWS), :])
    plsc.subcore_barrier()

    for j in range(_WROWS // _SROWS):
        pltpu.make_async_copy(
            shared, out_hbm.at[b, pl.ds(row0 + j * _SROWS, _SROWS), :], sem
        ).start()
    for j in range(_WROWS // _SROWS):
        pltpu.make_async_copy(
            shared, out_hbm.at[b, pl.ds(row0, _SROWS), :], sem
        ).wait()


_zero_fill = functools.partial(
    pl.kernel,
    out_type=jax.ShapeDtypeStruct((4, _ROWS // 4, _HID), jnp.float32),
    mesh=plsc.VectorSubcoreMesh(core_axis_name="c", subcore_axis_name="s"),
    scratch_types=[
        pltpu.VMEM((_BROWS, _HID), jnp.float32),
        pltpu.VMEM_SHARED((_SROWS, _HID), jnp.float32),
        pltpu.SemaphoreType.DMA,
    ],
)(_zero_fill_body)


def kernel(hidden_states, gate_weight):
    batch, seq, hidden = hidden_states.shape
    rows = batch * seq
    hs = hidden_states.reshape(rows, hidden)
    num_experts = gate_weight.shape[0]

    zeros = _zero_fill()

    logits = pl.pallas_call(
        _logits_body,
        grid=(rows // _BLOCK,),
        in_specs=[
            pl.BlockSpec((_BLOCK, hidden), lambda i: (i, 0)),
            pl.BlockSpec((num_experts, hidden), lambda i: (0, 0)),
        ],
        out_specs=pl.BlockSpec((_BLOCK, num_experts), lambda i: (i, 0)),
        out_shape=jax.ShapeDtypeStruct((rows, num_experts), jnp.float32),
    )(hs, gate_weight)

    return zeros, logits


# speedup vs baseline: 1.3189x; 1.3189x over previous
import jax
import jax.numpy as jnp
from jax.experimental import pallas as pl


_BLOCK = 2048


def _moe_gate_kernel(hs_ref, gw_ref, zero_ref, logits_ref):
    zero_ref[...] = jnp.zeros_like(zero_ref)
    logits_ref[...] = jax.lax.dot_general(
        gw_ref[...], hs_ref[...],
        dimension_numbers=(((1,), (1,)), ((), ())),
        preferred_element_type=jnp.float32,
    )


def kernel(hidden_states, gate_weight):
    batch, seq, hidden = hidden_states.shape
    rows = batch * seq
    hs = hidden_states.reshape(rows, hidden)
    num_experts = gate_weight.shape[0]

    zeros, logits_t = pl.pallas_call(
        _moe_gate_kernel,
        grid=(rows // _BLOCK,),
        in_specs=[
            pl.BlockSpec((_BLOCK, hidden), lambda i: (i, 0)),
            pl.BlockSpec((num_experts, hidden), lambda i: (0, 0)),
        ],
        out_specs=[
            pl.BlockSpec((_BLOCK, hidden), lambda i: (i, 0)),
            pl.BlockSpec((num_experts, _BLOCK), lambda i: (0, i)),
        ],
        out_shape=[
            jax.ShapeDtypeStruct((rows, hidden), hidden_states.dtype),
            jax.ShapeDtypeStruct((num_experts, rows), jnp.float32),
        ],
    )(hs, gate_weight)

    return zeros.reshape(batch, seq, hidden), logits_t.T
